# baseline (speedup 1.0000x reference)
import jax
import jax.numpy as jnp
from jax import lax
from jax.experimental import pallas as pl
from jax.experimental.pallas import tpu as pltpu
from jax.experimental.pallas import tpu_sc as plsc

_V = 50257
_D = 512
_T = 2048
_B = 4
_PAD_ID = 50256
_SCALE = float(_D) ** 0.5

_NUM_WORKERS = 32
_TW = _T // _NUM_WORKERS
_ST = 16
_NCHUNK = _TW // _ST
_ROWS = _B * _ST
_NBUF = 3
_LANES = 16


def _emb_body(ids_hbm, tok_hbm, pos_hbm, x_hbm, mask_hbm,
              ids_g, mask_v, pos_r, buf,
              idsem, gsems, psems, wsems):
    nc = plsc.get_sparse_core_info().num_cores
    wid = lax.axis_index("s") * nc + lax.axis_index("c")
    t0 = wid * _TW

    id_cps = [pltpu.make_async_copy(
        ids_hbm.at[pl.ds(b * _T + t0 + c * _ST, _ST)],
        ids_g.at[c, pl.ds(b * _ST, _ST)], idsem)
        for c in range(_NCHUNK) for b in range(_B)]
    for cp in id_cps:
        cp.start()

    def pos_cp(c):
        return pltpu.make_async_copy(
            pos_hbm.at[pl.ds(t0 + c * _ST, _ST)], pos_r.at[c % _NBUF],
            psems.at[c % _NBUF])

    def gather_cp(c):
        return pltpu.make_async_copy(
            tok_hbm.at[ids_g.at[c]], buf.at[c % _NBUF], gsems.at[c % _NBUF])

    def wb_cps(c):
        return [pltpu.make_async_copy(
            buf.at[c % _NBUF, pl.ds(b * _ST, _ST)],
            x_hbm.at[pl.ds(b * _T + t0 + c * _ST, _ST)],
            wsems.at[c % _NBUF]) for b in range(_B)]

    for c in range(_NBUF):
        pos_cp(c).start()
    for cp in id_cps:
        cp.wait()
    for c in range(2):
        gather_cp(c).start()

    for c in range(_NCHUNK):
        for half in range(_ROWS // _LANES):
            sl = pl.ds(half * _LANES, _LANES)
            v = ids_g[c, sl]
            mask_v[c, sl] = jnp.where(v != _PAD_ID, jnp.int32(1), jnp.int32(0))
    mask_cps = [pltpu.make_async_copy(
        mask_v.at[c, pl.ds(b * _ST, _ST)],
        mask_hbm.at[pl.ds(b * _T + t0 + c * _ST, _ST)], idsem)
        for c in range(_NCHUNK) for b in range(_B)]
    for cp in mask_cps:
        cp.start()

    for c in range(_NCHUNK):
        gather_cp(c).wait()
        pos_cp(c).wait()
        buf_c = buf.at[c % _NBUF]
        pos_c = pos_r.at[c % _NBUF]

        def row(t, _):
            for k in range(_D // _LANES):
                sl = pl.ds(k * _LANES, _LANES)
                pv = pos_c[t, sl]
                for b in range(_B):
                    r = b * _ST + t
                    buf_c[r, sl] = buf_c[r, sl] * _SCALE + pv
            return 0
        lax.fori_loop(0, _ST, row, 0)

        for cp in wb_cps(c):
            cp.start()
        if c + _NBUF < _NCHUNK:
            pos_cp(c + _NBUF).start()
        if c + 2 < _NCHUNK:
            if c - 1 >= 0:
                for cp in wb_cps(c - 1):
                    cp.wait()
            gather_cp(c + 2).start()

    for c in range(1, _NCHUNK):
        for cp in wb_cps(c):
            cp.wait()
    for cp in mask_cps:
        cp.wait()


@jax.jit
def _embed(ids_flat, tok_emb, pos_emb):
    mesh = plsc.VectorSubcoreMesh(core_axis_name="c", subcore_axis_name="s")
    f = pl.kernel(
        _emb_body,
        out_type=(
            jax.ShapeDtypeStruct((_B * _T, _D), jnp.float32),
            jax.ShapeDtypeStruct((_B * _T,), jnp.int32),
        ),
        mesh=mesh,
        scratch_types=[
            pltpu.VMEM((_NCHUNK, _ROWS), jnp.int32),
            pltpu.VMEM((_NCHUNK, _ROWS), jnp.int32),
            pltpu.VMEM((_NBUF, _ST, _D), jnp.float32),
            pltpu.VMEM((_NBUF, _ROWS, _D), jnp.float32),
            pltpu.SemaphoreType.DMA,
            pltpu.SemaphoreType.DMA((_NBUF,)),
            pltpu.SemaphoreType.DMA((_NBUF,)),
            pltpu.SemaphoreType.DMA((_NBUF,)),
        ],
    )
    return f(ids_flat, tok_emb, pos_emb)


def kernel(token_ids, tok_emb, pos_emb):
    B, T = token_ids.shape
    x_flat, mask_flat = _embed(token_ids.reshape(-1), tok_emb, pos_emb)
    x = x_flat.reshape(B, T, _D)
    attn_mask = mask_flat.reshape(B, T).astype(bool)[:, None, None, :]
    return (x, attn_mask)


# speedup vs baseline: 1.2861x; 1.2861x over previous
import jax
import jax.numpy as jnp
from jax import lax
from jax.experimental import pallas as pl
from jax.experimental.pallas import tpu as pltpu
from jax.experimental.pallas import tpu_sc as plsc

_V = 50257
_D = 512
_T = 2048
_B = 4
_PAD_ID = 50256
_SCALE = float(_D) ** 0.5

_NUM_WORKERS = 32
_TW = _T // _NUM_WORKERS
_ST = 8
_NCHUNK = _TW // _ST
_ROWS = _B * _ST
_NBUF = 4
_LANES = 16


def _emb_body(ids_hbm, tok_hbm, pos_hbm, x_hbm, mask_hbm,
              ids_g, mask_v, pos_v, buf,
              idsem, possem, gsems, wsems):
    nc = plsc.get_sparse_core_info().num_cores
    wid = lax.axis_index("s") * nc + lax.axis_index("c")
    t0 = wid * _TW

    id_cps = [pltpu.make_async_copy(
        ids_hbm.at[pl.ds(b * _T + t0 + c * _ST, _ST)],
        ids_g.at[c, pl.ds(b * _ST, _ST)], idsem)
        for c in range(_NCHUNK) for b in range(_B)]
    for cp in id_cps:
        cp.start()
    pos_cp = pltpu.async_copy(pos_hbm.at[pl.ds(t0, _TW)], pos_v, possem)
    for cp in id_cps:
        cp.wait()

    def gather_cp(c):
        return pltpu.make_async_copy(
            tok_hbm.at[ids_g.at[c]], buf.at[c % _NBUF], gsems.at[c % _NBUF])

    def wb_cps(c):
        return [pltpu.make_async_copy(
            buf.at[c % _NBUF, pl.ds(b * _ST, _ST)],
            x_hbm.at[pl.ds(b * _T + t0 + c * _ST, _ST)],
            wsems.at[c % _NBUF]) for b in range(_B)]

    for c in range(2):
        gather_cp(c).start()

    for c in range(_NCHUNK):
        for half in range(_ROWS // _LANES):
            sl = pl.ds(half * _LANES, _LANES)
            v = ids_g[c, sl]
            mask_v[c, sl] = jnp.where(v != _PAD_ID, jnp.int32(1), jnp.int32(0))
    mask_cps = [pltpu.make_async_copy(
        mask_v.at[c, pl.ds(b * _ST, _ST)],
        mask_hbm.at[pl.ds(b * _T + t0 + c * _ST, _ST)], idsem)
        for c in range(_NCHUNK) for b in range(_B)]
    for cp in mask_cps:
        cp.start()
    pos_cp.wait()

    for c in range(_NCHUNK):
        gather_cp(c).wait()
        buf_c = buf.at[c % _NBUF]

        def row(t, _):
            for k in range(_D // _LANES):
                sl = pl.ds(k * _LANES, _LANES)
                pv = pos_v[c * _ST + t, sl]
                for b in range(_B):
                    r = b * _ST + t
                    buf_c[r, sl] = buf_c[r, sl] * _SCALE + pv
            return 0
        lax.fori_loop(0, _ST, row, 0)

        for cp in wb_cps(c):
            cp.start()
        if c + 2 < _NCHUNK:
            if c >= 2:
                for cp in wb_cps(c - 2):
                    cp.wait()
            gather_cp(c + 2).start()

    for c in range(_NCHUNK - 4, _NCHUNK):
        for cp in wb_cps(c):
            cp.wait()
    for cp in mask_cps:
        cp.wait()


@jax.jit
def _embed(ids_flat, tok_emb, pos_emb):
    mesh = plsc.VectorSubcoreMesh(core_axis_name="c", subcore_axis_name="s")
    f = pl.kernel(
        _emb_body,
        out_type=(
            jax.ShapeDtypeStruct((_B * _T, _D), jnp.float32),
            jax.ShapeDtypeStruct((_B * _T,), jnp.int32),
        ),
        mesh=mesh,
        scratch_types=[
            pltpu.VMEM((_NCHUNK, _ROWS), jnp.int32),
            pltpu.VMEM((_NCHUNK, _ROWS), jnp.int32),
            pltpu.VMEM((_TW, _D), jnp.float32),
            pltpu.VMEM((_NBUF, _ROWS, _D), jnp.float32),
            pltpu.SemaphoreType.DMA,
            pltpu.SemaphoreType.DMA,
            pltpu.SemaphoreType.DMA((_NBUF,)),
            pltpu.SemaphoreType.DMA((_NBUF,)),
        ],
    )
    return f(ids_flat, tok_emb, pos_emb)


def kernel(token_ids, tok_emb, pos_emb):
    B, T = token_ids.shape
    x_flat, mask_flat = _embed(token_ids.reshape(-1), tok_emb, pos_emb)
    x = x_flat.reshape(B, T, _D)
    attn_mask = mask_flat.reshape(B, T).astype(bool)[:, None, None, :]
    return (x, attn_mask)


# speedup vs baseline: 1.3366x; 1.0392x over previous
import jax
import jax.numpy as jnp
from jax import lax
from jax.experimental import pallas as pl
from jax.experimental.pallas import tpu as pltpu
from jax.experimental.pallas import tpu_sc as plsc

_V = 50257
_D = 512
_T = 2048
_B = 4
_PAD_ID = 50256
_SCALE = float(_D) ** 0.5

_NUM_WORKERS = 32
_TW = _T // _NUM_WORKERS
_ST = 8
_NCHUNK = _TW // _ST
_NBUF = 4
_NROUND = _NCHUNK // _NBUF
_LANES = 16


def _emb_body(ids_hbm, tok_hbm, pos_hbm, x_hbm, mask_hbm,
              ids_v, mask_v, pos_v, buf,
              idsem, possem, gsems, wsems):
    nc = plsc.get_sparse_core_info().num_cores
    wid = lax.axis_index("s") * nc + lax.axis_index("c")
    t0 = wid * _TW

    id_cps = [pltpu.make_async_copy(ids_hbm.at[b, pl.ds(t0, _TW)],
                                    ids_v.at[b], idsem)
              for b in range(_B)]
    for cp in id_cps:
        cp.start()
    pos_cp = pltpu.async_copy(pos_hbm.at[pl.ds(t0, _TW)], pos_v, possem)
    for cp in id_cps:
        cp.wait()

    def gather_cps(c, i):
        return [pltpu.make_async_copy(
            tok_hbm.at[ids_v.at[b, pl.ds(c * _ST, _ST)]],
            buf.at[i, pl.ds(b * _ST, _ST)],
            gsems.at[i]) for b in range(_B)]

    def wb_cps(c, i):
        return [pltpu.make_async_copy(
            buf.at[i, pl.ds(b * _ST, _ST)],
            x_hbm.at[pl.ds(b * _T + t0 + c * _ST, _ST)],
            wsems.at[i]) for b in range(_B)]

    for c in range(2):
        for cp in gather_cps(c, c):
            cp.start()

    for b in range(_B):
        def mask_vec(k, _):
            sl = pl.ds(k * _LANES, _LANES)
            v = ids_v[b, sl]
            mask_v[b, sl] = jnp.where(v != _PAD_ID, jnp.int32(1), jnp.int32(0))
            return 0
        lax.fori_loop(0, _TW // _LANES, mask_vec, 0)
        pltpu.sync_copy(mask_v.at[b], mask_hbm.at[pl.ds(b * _T + t0, _TW)])
    pos_cp.wait()

    def round_body(g, _):
        for i in range(_NBUF):
            c = g * _NBUF + i
            for cp in gather_cps(c, i):
                cp.wait()
            buf_i = buf.at[i]

            def row(t, _):
                for k in range(_D // _LANES):
                    sl = pl.ds(k * _LANES, _LANES)
                    pv = pos_v[c * _ST + t, sl]
                    for b in range(_B):
                        r = b * _ST + t
                        buf_i[r, sl] = buf_i[r, sl] * _SCALE + pv
                return 0
            lax.fori_loop(0, _ST, row, 0)

            for cp in wb_cps(c, i):
                cp.start()

            j = (i + 2) % _NBUF

            @pl.when(c + 2 < _NCHUNK)
            def _prefetch():
                @pl.when(c >= 2)
                def _drain():
                    for cp in wb_cps(c - 2, j):
                        cp.wait()
                for cp in gather_cps(c + 2, j):
                    cp.start()
        return 0
    lax.fori_loop(0, _NROUND, round_body, 0)

    for c in range(_NCHUNK - 2, _NCHUNK):
        for cp in wb_cps(c, c % _NBUF):
            cp.wait()


@jax.jit
def _embed(token_ids, tok_emb, pos_emb):
    mesh = plsc.VectorSubcoreMesh(core_axis_name="c", subcore_axis_name="s")
    f = pl.kernel(
        _emb_body,
        out_type=(
            jax.ShapeDtypeStruct((_B * _T, _D), jnp.float32),
            jax.ShapeDtypeStruct((_B * _T,), jnp.int32),
        ),
        mesh=mesh,
        scratch_types=[
            pltpu.VMEM((_B, _TW), jnp.int32),
            pltpu.VMEM((_B, _TW), jnp.int32),
            pltpu.VMEM((_TW, _D), jnp.float32),
            pltpu.VMEM((_NBUF, _B * _ST, _D), jnp.float32),
            pltpu.SemaphoreType.DMA,
            pltpu.SemaphoreType.DMA,
            pltpu.SemaphoreType.DMA((_NBUF,)),
            pltpu.SemaphoreType.DMA((_NBUF,)),
        ],
    )
    return f(token_ids, tok_emb, pos_emb)


def kernel(token_ids, tok_emb, pos_emb):
    B, T = token_ids.shape
    x_flat, mask_flat = _embed(token_ids, tok_emb, pos_emb)
    x = x_flat.reshape(B, T, _D)
    attn_mask = mask_flat.reshape(B, T).astype(bool)[:, None, None, :]
    return (x, attn_mask)


# speedup vs baseline: 1.3376x; 1.0007x over previous
import jax
import jax.numpy as jnp
from jax import lax
from jax.experimental import pallas as pl
from jax.experimental.pallas import tpu as pltpu
from jax.experimental.pallas import tpu_sc as plsc

_V = 50257
_D = 512
_T = 2048
_B = 4
_PAD_ID = 50256
_SCALE = float(_D) ** 0.5

_NUM_WORKERS = 32
_TW = _T // _NUM_WORKERS
_ST = 8
_NCHUNK = _TW // _ST
_NBUF = 4
_NROUND = _NCHUNK // _NBUF
_LANES = 16


def _emb_body(ids_hbm, tok_hbm, pos_hbm, x_hbm, mask_hbm,
              ids_v, mask_v, pos_v, buf,
              idsem, possem, gsems, wsems):
    nc = plsc.get_sparse_core_info().num_cores
    wid = lax.axis_index("s") * nc + lax.axis_index("c")
    t0 = wid * _TW

    id_cps = [pltpu.make_async_copy(ids_hbm.at[b, pl.ds(t0, _TW)],
                                    ids_v.at[b], idsem)
              for b in range(_B)]
    for cp in id_cps:
        cp.start()
    pos_cp = pltpu.async_copy(pos_hbm.at[pl.ds(t0, _TW)], pos_v, possem)
    for cp in id_cps:
        cp.wait()

    def gather_cps(c, i):
        return [pltpu.make_async_copy(
            tok_hbm.at[ids_v.at[b, pl.ds(c * _ST, _ST)]],
            buf.at[i, pl.ds(b * _ST, _ST)],
            gsems.at[i]) for b in range(_B)]

    def wb_cps(c, i):
        return [pltpu.make_async_copy(
            buf.at[i, pl.ds(b * _ST, _ST)],
            x_hbm.at[pl.ds(b * _T + t0 + c * _ST, _ST)],
            wsems.at[i]) for b in range(_B)]

    for c in range(2):
        for cp in gather_cps(c, c):
            cp.start()

    for b in range(_B):
        def mask_vec(k, _):
            sl = pl.ds(k * _LANES, _LANES)
            v = ids_v[b, sl]
            mask_v[b, sl] = jnp.where(v != _PAD_ID, jnp.int32(1), jnp.int32(0))
            return 0
        lax.fori_loop(0, _TW // _LANES, mask_vec, 0)
        pltpu.sync_copy(mask_v.at[b], mask_hbm.at[b, 0, 0, pl.ds(t0, _TW)])
    pos_cp.wait()

    def round_body(g, _):
        for i in range(_NBUF):
            c = g * _NBUF + i
            for cp in gather_cps(c, i):
                cp.wait()
            buf_i = buf.at[i]

            def row(t, _):
                for k in range(_D // _LANES):
                    sl = pl.ds(k * _LANES, _LANES)
                    pv = pos_v[c * _ST + t, sl]
                    for b in range(_B):
                        r = b * _ST + t
                        buf_i[r, sl] = buf_i[r, sl] * _SCALE + pv
                return 0
            lax.fori_loop(0, _ST, row, 0)

            for cp in wb_cps(c, i):
                cp.start()

            j = (i + 2) % _NBUF

            @pl.when(c + 2 < _NCHUNK)
            def _prefetch():
                @pl.when(c >= 2)
                def _drain():
                    for cp in wb_cps(c - 2, j):
                        cp.wait()
                for cp in gather_cps(c + 2, j):
                    cp.start()
        return 0
    lax.fori_loop(0, _NROUND, round_body, 0)

    for c in range(_NCHUNK - 2, _NCHUNK):
        for cp in wb_cps(c, c % _NBUF):
            cp.wait()


@jax.jit
def _embed(token_ids, tok_emb, pos_emb):
    mesh = plsc.VectorSubcoreMesh(core_axis_name="c", subcore_axis_name="s")
    f = pl.kernel(
        _emb_body,
        out_type=(
            jax.ShapeDtypeStruct((_B * _T, _D), jnp.float32),
            jax.ShapeDtypeStruct((_B, 1, 1, _T), jnp.int32),
        ),
        mesh=mesh,
        scratch_types=[
            pltpu.VMEM((_B, _TW), jnp.int32),
            pltpu.VMEM((_B, _TW), jnp.int32),
            pltpu.VMEM((_TW, _D), jnp.float32),
            pltpu.VMEM((_NBUF, _B * _ST, _D), jnp.float32),
            pltpu.SemaphoreType.DMA,
            pltpu.SemaphoreType.DMA,
            pltpu.SemaphoreType.DMA((_NBUF,)),
            pltpu.SemaphoreType.DMA((_NBUF,)),
        ],
    )
    return f(token_ids, tok_emb, pos_emb)


def kernel(token_ids, tok_emb, pos_emb):
    B, T = token_ids.shape
    x_flat, mask_i32 = _embed(token_ids, tok_emb, pos_emb)
    x = x_flat.reshape(B, T, _D)
    attn_mask = mask_i32.astype(bool)
    return (x, attn_mask)
